# baseline (speedup 1.0000x reference)
import jax
import jax.numpy as jnp
from jax.experimental import pallas as pl
from jax.experimental.pallas import tpu as pltpu

_NOBJ = 64
_V = 262144


def _body(obj_sm, w_hbm, o_hbm, sem):
    obj = obj_sm[0]
    copy = pltpu.make_async_copy(w_hbm.at[obj], o_hbm.at[0], sem)
    copy.start()
    copy.wait()


def kernel(obj_id, weights):
    obj = jnp.asarray(obj_id, dtype=jnp.int32).reshape(1)
    return pl.pallas_call(
        _body,
        in_specs=[
            pl.BlockSpec(memory_space=pltpu.SMEM),
            pl.BlockSpec(memory_space=pl.ANY),
        ],
        out_specs=pl.BlockSpec(memory_space=pl.ANY),
        out_shape=jax.ShapeDtypeStruct((1, _V, 3), jnp.float32),
        scratch_shapes=[pltpu.SemaphoreType.DMA],
    )(obj, weights)


# speedup vs baseline: 6.2674x; 6.2674x over previous
import jax
import jax.numpy as jnp
from jax.experimental import pallas as pl
from jax.experimental.pallas import tpu as pltpu

_NOBJ = 64
_V = 262144
_R = (_V * 3) // 128


def _body(obj_sm, w_hbm, o_hbm, sem):
    obj = obj_sm[0]
    copy = pltpu.make_async_copy(w_hbm.at[obj], o_hbm.at[0], sem)
    copy.start()
    copy.wait()


def kernel(obj_id, weights):
    obj = jnp.asarray(obj_id, dtype=jnp.int32).reshape(1)
    w = weights.reshape(_NOBJ, _R, 128)
    out = pl.pallas_call(
        _body,
        in_specs=[
            pl.BlockSpec(memory_space=pltpu.SMEM),
            pl.BlockSpec(memory_space=pl.ANY),
        ],
        out_specs=pl.BlockSpec(memory_space=pl.ANY),
        out_shape=jax.ShapeDtypeStruct((1, _R, 128), jnp.float32),
        scratch_shapes=[pltpu.SemaphoreType.DMA],
    )(obj, w)
    return out.reshape(1, _V, 3)
